# baseline (speedup 1.0000x reference)
import functools
import jax
import jax.numpy as jnp
from jax import lax
from jax.experimental import pallas as pl
from jax.experimental.pallas import tpu as pltpu
from jax.experimental.pallas import tpu_sc as plsc

_HEADS = 8
_NG = 64

_N = 10000
_E = 320000
_NROW = 10016
_NW = 32
_K = 128
_C = 80
_EPAD = _NW * _C * _K
_RPT = 632
_ACC = _NW // 2 * _RPT

_mesh = plsc.VectorSubcoreMesh(core_axis_name="c", subcore_axis_name="s")


def _make_seg_kernel(width):
    out_t = jax.ShapeDtypeStruct((2, _ACC, width), jnp.float32)

    @functools.partial(
        pl.kernel, mesh=_mesh, out_type=out_t,
        scratch_types=[
            pltpu.VMEM((2, _K), jnp.int32),
            pltpu.VMEM((2, _K), jnp.int32),
            pltpu.VMEM((_K, width), jnp.float32),
            pltpu.VMEM((_K, width), jnp.float32),
            pltpu.VMEM_SHARED((_ACC, width), jnp.float32),
            pltpu.SemaphoreType.DMA,
            pltpu.SemaphoreType.DMA,
            pltpu.SemaphoreType.DMA,
        ],
    )
    def k(table, sidx, didx, zeros, out, sidxc, didxc, buf0, buf1, acc,
          semg0, semg1, semi):
        c = lax.axis_index("c")
        s = lax.axis_index("s")
        w = c * 16 + s
        bufs = (buf0, buf1)
        semgs = (semg0, semg1)
        pltpu.sync_copy(zeros, acc.at[pl.ds(s * _RPT, _RPT)])
        for b in range(2):
            pltpu.sync_copy(sidx.at[w, b, 0], sidxc.at[b])
            pltpu.sync_copy(didx.at[w, b, 0], didxc.at[b])
        plsc.subcore_barrier()
        for b in range(2):
            pltpu.async_copy(table.at[sidxc.at[b]], bufs[b], semgs[b])

        def pair(j2, carry):
            for b in range(2):
                j = j2 * 2 + b
                pltpu.make_async_copy(table.at[sidxc.at[b]], bufs[b],
                                      semgs[b]).wait()

                @pl.when(j < _C - 2)
                def _():
                    cpi1 = pltpu.async_copy(sidx.at[w, j + 2, 0],
                                            sidxc.at[b], semi)
                    pltpu.sync_copy(bufs[b], acc.at[didxc.at[b]], add=True)
                    cpi2 = pltpu.async_copy(didx.at[w, j + 2, 0],
                                            didxc.at[b], semi)
                    cpi1.wait()
                    cpi2.wait()
                    pltpu.async_copy(table.at[sidxc.at[b]], bufs[b], semgs[b])

                @pl.when(j >= _C - 2)
                def _():
                    pltpu.sync_copy(bufs[b], acc.at[didxc.at[b]], add=True)
            return carry

        lax.fori_loop(0, _C // 2, pair, 0)
        plsc.subcore_barrier()
        pltpu.sync_copy(acc.at[pl.ds(s * _RPT, _RPT)],
                        out.at[c, pl.ds(s * _RPT, _RPT)])

    return k


def _splat(v, i):
    idx = jnp.full((16, 1), i, jnp.int32)
    dn = lax.GatherDimensionNumbers(
        offset_dims=(), collapsed_slice_dims=(0,), start_index_map=(0,))
    return lax.gather(v, idx, dn, (1,),
                      mode=lax.GatherScatterMode.PROMISE_IN_BOUNDS)


def _idx_build(sidx_v, didx_v, sidxb, didxb, j):
    def tgrp(t, cc):
        sv = sidx_v[j, pl.ds(t * 16, 16)] * _HEADS
        dv = didx_v[j, pl.ds(t * 16, 16)] * _HEADS
        for h in range(_HEADS):
            sidxb[h, pl.ds(t * 16, 16)] = sv + h
            didxb[h, pl.ds(t * 16, 16)] = dv + h
        return cc

    lax.fori_loop(0, _K // 16, tgrp, 0)


def _make_gatf_kernel():
    out_t = jax.ShapeDtypeStruct((2, _ACC, 128), jnp.float32)

    @functools.partial(
        pl.kernel, mesh=_mesh, out_type=out_t,
        scratch_types=[
            pltpu.VMEM((2, _K), jnp.int32),
            pltpu.VMEM((2, _K), jnp.int32),
            pltpu.VMEM((2 * _HEADS, _K), jnp.int32),
            pltpu.VMEM((2 * _HEADS, _K), jnp.int32),
            pltpu.VMEM((_K, 128), jnp.float32),
            pltpu.VMEM((_K, 128), jnp.float32),
            pltpu.VMEM((2 * _HEADS, _K), jnp.float32),
            pltpu.VMEM((2 * _HEADS, _K), jnp.float32),
            pltpu.VMEM_SHARED((_ACC, 128), jnp.float32),
            pltpu.SemaphoreType.DMA,
            pltpu.SemaphoreType.DMA,
            pltpu.SemaphoreType.DMA,
            pltpu.SemaphoreType.DMA,
            pltpu.SemaphoreType.DMA,
        ],
    )
    def k(hgat, asrcf, adstf, sidx, didx, zeros128, outf,
          sidxc, didxc, sidxb, didxb, hbuf0, hbuf1, sabuf, dabuf,
          accf, semh0, semh1, sema0, sema1, semi):
        c = lax.axis_index("c")
        s = lax.axis_index("s")
        w = c * 16 + s
        hbufs = (hbuf0, hbuf1)
        semhs = (semh0, semh1)
        semas = (sema0, sema1)
        pltpu.sync_copy(zeros128, accf.at[pl.ds(s * _RPT, _RPT)])
        for b in range(2):
            pltpu.sync_copy(sidx.at[w, b, 0], sidxc.at[b])
            pltpu.sync_copy(didx.at[w, b, 0], didxc.at[b])
        plsc.subcore_barrier()

        def build_and_launch(b):
            pltpu.async_copy(hgat.at[sidxc.at[b]], hbufs[b], semhs[b])

            def tgrp(t, cc):
                sv = sidxc[b, pl.ds(t * 16, 16)] * _HEADS
                dv = didxc[b, pl.ds(t * 16, 16)] * _HEADS
                for h in range(_HEADS):
                    sidxb[b * _HEADS + h, pl.ds(t * 16, 16)] = sv + h
                    didxb[b * _HEADS + h, pl.ds(t * 16, 16)] = dv + h
                return cc

            lax.fori_loop(0, _K // 16, tgrp, 0)
            for h in range(_HEADS):
                pltpu.async_copy(asrcf.at[sidxb.at[b * _HEADS + h]],
                                 sabuf.at[b * _HEADS + h], semas[b])
                pltpu.async_copy(adstf.at[didxb.at[b * _HEADS + h]],
                                 dabuf.at[b * _HEADS + h], semas[b])

        for b in range(2):
            build_and_launch(b)

        def pair(j2, carry):
            for b in range(2):
                j = j2 * 2 + b
                pltpu.make_async_copy(hgat.at[sidxc.at[b]], hbufs[b],
                                      semhs[b]).wait()
                for h in range(_HEADS):
                    pltpu.make_async_copy(
                        asrcf.at[sidxb.at[b * _HEADS + h]],
                        sabuf.at[b * _HEADS + h], semas[b]).wait()
                    pltpu.make_async_copy(
                        adstf.at[didxb.at[b * _HEADS + h]],
                        dabuf.at[b * _HEADS + h], semas[b]).wait()

                def grp(t, cc):
                    evs = []
                    for h in range(_HEADS):
                        a = (sabuf[b * _HEADS + h, pl.ds(t * 16, 16)]
                             + dabuf[b * _HEADS + h, pl.ds(t * 16, 16)])
                        a = jnp.where(a > 0, a, a * 0.2)
                        evs.append(jnp.exp(a))
                    for i in range(16):
                        e = t * 16 + i
                        for h in range(_HEADS):
                            sp = _splat(evs[h], i)
                            hbufs[b][e, pl.ds(16 * h, 16)] = (
                                hbufs[b][e, pl.ds(16 * h, 16)] * sp)
                    return cc

                lax.fori_loop(0, _K // 16, grp, 0)

                @pl.when(j < _C - 2)
                def _():
                    cpi1 = pltpu.async_copy(sidx.at[w, j + 2, 0],
                                            sidxc.at[b], semi)
                    pltpu.sync_copy(hbufs[b], accf.at[didxc.at[b]], add=True)
                    cpi2 = pltpu.async_copy(didx.at[w, j + 2, 0],
                                            didxc.at[b], semi)
                    cpi1.wait()
                    cpi2.wait()
                    build_and_launch(b)

                @pl.when(j >= _C - 2)
                def _():
                    pltpu.sync_copy(hbufs[b], accf.at[didxc.at[b]], add=True)
            return carry

        lax.fori_loop(0, _C // 2, pair, 0)
        plsc.subcore_barrier()
        pltpu.sync_copy(accf.at[pl.ds(s * _RPT, _RPT)],
                        outf.at[c, pl.ds(s * _RPT, _RPT)])

    return k


def _make_gatz_kernel():
    out_t = [jax.ShapeDtypeStruct((2 * _ACC * _HEADS,), jnp.float32),
             jax.ShapeDtypeStruct((2 * _ACC,), jnp.float32)]

    @functools.partial(
        pl.kernel, mesh=_mesh, out_type=out_t,
        scratch_types=[
            pltpu.VMEM((_C, _K), jnp.int32),
            pltpu.VMEM((_C, _K), jnp.int32),
            pltpu.VMEM((_HEADS, _K), jnp.int32),
            pltpu.VMEM((_HEADS, _K), jnp.int32),
            pltpu.VMEM((_K,), jnp.int32),
            pltpu.VMEM((_K,), jnp.float32),
            pltpu.VMEM((_HEADS, _K), jnp.float32),
            pltpu.VMEM((_HEADS, _K), jnp.float32),
            pltpu.VMEM((_HEADS, _K), jnp.float32),
            pltpu.VMEM((_RPT * _HEADS,), jnp.float32),
            pltpu.VMEM_SHARED((_ACC * _HEADS,), jnp.float32),
            pltpu.VMEM_SHARED((_ACC,), jnp.float32),
            pltpu.SemaphoreType.DMA,
        ],
    )
    def k(asrcf, adstf, sidx, didx, zerosz, oute, outd,
          sidx_v, didx_v, sidxb, didxb, didxp, onesb, sabuf, dabuf, evbuf,
          zbuf, acce, accd, sema):
        c = lax.axis_index("c")
        s = lax.axis_index("s")
        w = c * 16 + s
        zr = _RPT * _HEADS
        pltpu.sync_copy(sidx.at[w], sidx_v)
        pltpu.sync_copy(didx.at[w], didx_v)
        pltpu.sync_copy(zerosz, zbuf)
        pltpu.sync_copy(zbuf, acce.at[pl.ds(s * zr, zr)])
        pltpu.sync_copy(zbuf.at[pl.ds(0, _RPT)], accd.at[pl.ds(s * _RPT, _RPT)])
        for t in range(_K // 16):
            onesb[pl.ds(t * 16, 16)] = jnp.ones((16,), jnp.float32)
        plsc.subcore_barrier()

        def chunk(j, carry):
            def tgrp(t, cc):
                sv = sidx_v[j, pl.ds(t * 16, 16)] * _HEADS
                dvp = didx_v[j, pl.ds(t * 16, 16)]
                didxp[pl.ds(t * 16, 16)] = dvp
                dv = dvp * _HEADS
                for h in range(_HEADS):
                    sidxb[h, pl.ds(t * 16, 16)] = sv + h
                    didxb[h, pl.ds(t * 16, 16)] = dv + h
                return cc

            lax.fori_loop(0, _K // 16, tgrp, 0)
            cps = [pltpu.async_copy(asrcf.at[sidxb.at[h]], sabuf.at[h], sema)
                   for h in range(_HEADS)]
            cpd = [pltpu.async_copy(adstf.at[didxb.at[h]], dabuf.at[h], sema)
                   for h in range(_HEADS)]
            pltpu.sync_copy(onesb, accd.at[didxp], add=True)
            for cp in cps:
                cp.wait()
            for cp in cpd:
                cp.wait()

            def grp(t, cc):
                for h in range(_HEADS):
                    a = sabuf[h, pl.ds(t * 16, 16)] + dabuf[h, pl.ds(t * 16, 16)]
                    a = jnp.where(a > 0, a, a * 0.2)
                    evbuf[h, pl.ds(t * 16, 16)] = jnp.exp(a)
                return cc

            lax.fori_loop(0, _K // 16, grp, 0)
            for h in range(_HEADS):
                pltpu.sync_copy(evbuf.at[h], acce.at[didxb.at[h]], add=True)
            return carry

        lax.fori_loop(0, _C, chunk, 0)
        plsc.subcore_barrier()
        pltpu.sync_copy(acce.at[pl.ds(s * zr, zr)], zbuf)
        pltpu.sync_copy(zbuf, oute.at[pl.ds(c * (_ACC * _HEADS) + s * zr, zr)])
        pltpu.sync_copy(accd.at[pl.ds(s * _RPT, _RPT)], zbuf.at[pl.ds(0, _RPT)])
        pltpu.sync_copy(zbuf.at[pl.ds(0, _RPT)],
                        outd.at[pl.ds(c * _ACC + s * _RPT, _RPT)])

    return k


_seg128_kernel = _make_seg_kernel(128)
_gatf_kernel = _make_gatf_kernel()
_gatz_kernel = _make_gatz_kernel()


def _bn(x, g, b, eps=1e-5):
    mu = jnp.mean(x, axis=0)
    var = jnp.var(x, axis=0)
    return g * (x - mu) / jnp.sqrt(var + eps) + b


def _leaky(a):
    return jnp.where(a > 0, a, 0.2 * a)


def _pad_rows(a):
    return jnp.concatenate(
        [a, jnp.zeros((_NROW - _N, a.shape[1]), a.dtype)], axis=0)


def _gat_dense(x, Wg, a_s, a_d):
    h = x @ Wg
    hr = h.reshape(_N, _HEADS, -1)
    asrc = jnp.sum(hr * a_s, axis=-1)
    adst = jnp.sum(hr * a_d, axis=-1)
    return h, asrc, adst


def _gat_combine(num, z, h, asrc, adst, bg):
    eself = jnp.exp(_leaky(asrc + adst))
    ch = h.shape[1] // _HEADS
    numr = num.reshape(_N, _HEADS, ch) + eself[:, :, None] * h.reshape(_N, _HEADS, ch)
    zr = z + eself
    out = numr / (zr[:, :, None] + 1e-16)
    return out.reshape(_N, -1) + bg


def _run_gat(x, Wg, a_s, a_d, bg, sidx4, didx4, sidx, didx, zeros128, zerosz):
    h, asrc, adst = _gat_dense(x, Wg, a_s, a_d)
    hgat = _pad_rows(h)
    asrcf = _pad_rows(asrc).reshape(-1)
    adstf = _pad_rows(adst).reshape(-1)
    pf = _gatf_kernel(hgat, asrcf, adstf, sidx4, didx4, zeros128)
    pe, pdeg = _gatz_kernel(asrcf, adstf, sidx, didx, zerosz)
    num = pf[0, :_N] + pf[1, :_N]
    pe = pe.reshape(2, _ACC, _HEADS)
    z = pe[0, :_N] + pe[1, :_N]
    pdeg = pdeg.reshape(2, _ACC)
    indeg = pdeg[0, :_N] + pdeg[1, :_N]
    return _gat_combine(num, z, h, asrc, adst, bg), indeg


def _run_seg(table, sidx4, didx4, zeros):
    width = table.shape[1]
    if width < 128:
        table = jnp.concatenate(
            [table, jnp.zeros((_N, 128 - width), jnp.float32)], axis=1)
    parts = _seg128_kernel(_pad_rows(table), sidx4, didx4, zeros)
    return parts[0, :_N, :width] + parts[1, :_N, :width]


def _head_body(x_ref, batch_ref, *refs):
    (Wp, bp, Wa, ba, Wl1, bl1, gl1, bel1, Wl2, bl2, gl2, bel2,
     Wl3, bl3, Wl4, bl4, out_ref, xmax_ref) = refs
    x = x_ref[...]
    n = x.shape[0]
    b = batch_ref[...]
    gid = lax.broadcasted_iota(jnp.int32, (n, _NG), 1)
    oh = (b == gid).astype(jnp.float32)
    xsum = lax.dot_general(oh, x, (((0,), (0,)), ((), ())),
                           preferred_element_type=jnp.float32)
    cnt = jnp.sum(oh, axis=0)
    xmean = xsum / jnp.maximum(cnt, 1.0)[:, None]

    def gmax(g, carry):
        m = jnp.where(b == g, x, -jnp.inf)
        mg = jnp.max(m, axis=0)
        xmax_ref[pl.ds(g, 1), :] = mg[None, :]
        return carry

    lax.fori_loop(0, _NG, gmax, 0)
    xmax = xmax_ref[...]
    xmax = jnp.where(xmax > -1e30, xmax, 0.0)

    xp = jnp.concatenate([xmean, xmax, xsum], axis=1)
    xp = jax.nn.relu(xp @ Wp[...] + bp[...])
    att = jax.nn.sigmoid(xp @ Wa[...] + ba[...])
    xp = xp * att
    h = jax.nn.relu(xp @ Wl1[...] + bl1[...])
    h = _bn(h, gl1[...], bel1[...])
    h = jax.nn.relu(h @ Wl2[...] + bl2[...])
    h = _bn(h, gl2[...], bel2[...])
    h = jax.nn.relu(h @ Wl3[...] + bl3[...])
    h = h @ Wl4[...] + bl4[...]
    out_ref[...] = jax.nn.log_softmax(h, axis=1)


def _head(x4, batch, p):
    n = x4.shape[0]
    args = (x4, batch.astype(jnp.int32).reshape(n, 1),
            p['Wp'], p['bp'].reshape(1, -1), p['Wa'], p['ba'].reshape(1, -1),
            p['Wl1'], p['bl1'].reshape(1, -1), p['gl1'].reshape(1, -1),
            p['bel1'].reshape(1, -1),
            p['Wl2'], p['bl2'].reshape(1, -1), p['gl2'].reshape(1, -1),
            p['bel2'].reshape(1, -1),
            p['Wl3'], p['bl3'].reshape(1, -1), p['Wl4'], p['bl4'].reshape(1, -1))
    return pl.pallas_call(
        _head_body,
        out_shape=jax.ShapeDtypeStruct((_NG, 6), jnp.float32),
        scratch_shapes=[pltpu.VMEM((_NG, x4.shape[1]), jnp.float32)],
    )(*args)


def kernel(x, edge_index, batch, params):
    p = params

    pad = jnp.full((_EPAD - _E,), _N, jnp.int32)
    sidx = jnp.concatenate([edge_index[0].astype(jnp.int32), pad]).reshape(_NW, _C, _K)
    didx = jnp.concatenate([edge_index[1].astype(jnp.int32), pad]).reshape(_NW, _C, _K)
    sidx4 = sidx.reshape(_NW, _C, 1, _K)
    didx4 = didx.reshape(_NW, _C, 1, _K)
    zeros128 = jnp.zeros((_RPT, 128), jnp.float32)
    zerosz = jnp.zeros((_RPT * _HEADS,), jnp.float32)

    x0 = x @ p['W_in'] + p['b_in']
    x0 = jax.nn.relu(_bn(x0, p['g_in'], p['be_in']))

    xg1, indeg = _run_gat(x0, p['Wg1'], p['as1'], p['ad1'], p['bg1'],
                          sidx4, didx4, sidx, didx, zeros128, zerosz)
    dinv = (indeg + 1.0) ** -0.5
    xg1 = jax.nn.elu(_bn(xg1, p['g1'], p['be1']))
    hc1 = dinv[:, None] * (x0 @ p['Wc1'])
    S1 = _run_seg(hc1, sidx4, didx4, zeros128)
    xc1 = dinv[:, None] * (S1 + hc1) + p['bc1']
    xc1 = jax.nn.elu(_bn(xc1, p['gc1'], p['bec1']))
    x1 = xg1 + xc1 + x0

    xg2, _unused = _run_gat(x1, p['Wg2'], p['as2'], p['ad2'], p['bg2'],
                            sidx4, didx4, sidx, didx, zeros128, zerosz)
    xg2 = jax.nn.elu(_bn(xg2, p['g2'], p['be2']))
    hc2 = dinv[:, None] * (x1 @ p['Wc2'])
    S2 = _run_seg(hc2, sidx4, didx4, zeros128)
    xc2 = dinv[:, None] * (S2 + hc2) + p['bc2']
    xc2 = jax.nn.elu(_bn(xc2, p['gc2'], p['bec2']))
    x2 = xg2 + xc2 + x1

    agg = _run_seg(x2, sidx4, didx4, zeros128)
    x3 = agg @ p['Wr3'] + p['br3'] + x2 @ p['Wroot3']
    x3 = jax.nn.elu(_bn(x3, p['g3'], p['be3']))
    residual = x1 @ p['Wres'] + p['bres']

    hc4 = dinv[:, None] * (x3 @ p['W4'])
    S4 = _run_seg(hc4, sidx4, didx4, zeros128)
    x4 = dinv[:, None] * (S4 + hc4) + p['b4']
    x4 = jax.nn.elu(_bn(x4, p['g4'], p['be4']))
    x4 = x4 + residual

    return _head(x4, batch, p)


# speedup vs baseline: 1.2896x; 1.2896x over previous
import functools
import jax
import jax.numpy as jnp
from jax import lax
from jax.experimental import pallas as pl
from jax.experimental.pallas import tpu as pltpu
from jax.experimental.pallas import tpu_sc as plsc

_HEADS = 8
_NG = 64

_N = 10000
_E = 320000
_NROW = 10016
_NW = 32
_K = 128
_C = 79
_EPAD = _NW * _C * _K
_RPT = 632
_ACC = _NW // 2 * _RPT

_mesh = plsc.VectorSubcoreMesh(core_axis_name="c", subcore_axis_name="s")


def _make_seg_kernel(width):
    out_t = jax.ShapeDtypeStruct((2, _ACC, width), jnp.float32)

    @functools.partial(
        pl.kernel, mesh=_mesh, out_type=out_t,
        scratch_types=[
            pltpu.VMEM((_C, _K), jnp.int32),
            pltpu.VMEM((_C, _K), jnp.int32),
            pltpu.VMEM((_K, width), jnp.float32),
            pltpu.VMEM_SHARED((_ACC, width), jnp.float32),
            pltpu.SemaphoreType.DMA,
        ],
    )
    def k(table, sidx, didx, zeros, out, sidx_v, didx_v, buf, acc, sem):
        c = lax.axis_index("c")
        s = lax.axis_index("s")
        w = c * 16 + s
        pltpu.sync_copy(sidx.at[w], sidx_v)
        pltpu.sync_copy(didx.at[w], didx_v)
        pltpu.sync_copy(zeros, acc.at[pl.ds(s * _RPT, _RPT)])
        plsc.subcore_barrier()

        def chunk(j, carry):
            pltpu.async_copy(table.at[sidx_v.at[j]], buf, sem).wait()
            pltpu.sync_copy(buf, acc.at[didx_v.at[j]], add=True)
            return carry

        lax.fori_loop(0, _C, chunk, 0)
        plsc.subcore_barrier()
        pltpu.sync_copy(acc.at[pl.ds(s * _RPT, _RPT)],
                        out.at[c, pl.ds(s * _RPT, _RPT)])

    return k


def _splat(v, i):
    idx = jnp.full((16, 1), i, jnp.int32)
    dn = lax.GatherDimensionNumbers(
        offset_dims=(), collapsed_slice_dims=(0,), start_index_map=(0,))
    return lax.gather(v, idx, dn, (1,),
                      mode=lax.GatherScatterMode.PROMISE_IN_BOUNDS)


def _idx_build(sidx_v, didx_v, sidxb, didxb, j):
    def tgrp(t, cc):
        sv = sidx_v[j, pl.ds(t * 16, 16)] * _HEADS
        dv = didx_v[j, pl.ds(t * 16, 16)] * _HEADS
        for h in range(_HEADS):
            sidxb[h, pl.ds(t * 16, 16)] = sv + h
            didxb[h, pl.ds(t * 16, 16)] = dv + h
        return cc

    lax.fori_loop(0, _K // 16, tgrp, 0)


def _make_gatf_kernel():
    out_t = jax.ShapeDtypeStruct((2, _ACC, 128), jnp.float32)

    @functools.partial(
        pl.kernel, mesh=_mesh, out_type=out_t,
        scratch_types=[
            pltpu.VMEM((_C, _K), jnp.int32),
            pltpu.VMEM((_C, _K), jnp.int32),
            pltpu.VMEM((_HEADS, _K), jnp.int32),
            pltpu.VMEM((_HEADS, _K), jnp.int32),
            pltpu.VMEM((_K, 128), jnp.float32),
            pltpu.VMEM((_HEADS, _K), jnp.float32),
            pltpu.VMEM((_HEADS, _K), jnp.float32),
            pltpu.VMEM_SHARED((_ACC, 128), jnp.float32),
            pltpu.SemaphoreType.DMA,
            pltpu.SemaphoreType.DMA,
        ],
    )
    def k(hgat, asrcf, adstf, sidx, didx, zeros128, outf,
          sidx_v, didx_v, sidxb, didxb, hbuf, sabuf, dabuf,
          accf, semh, sema):
        c = lax.axis_index("c")
        s = lax.axis_index("s")
        w = c * 16 + s
        pltpu.sync_copy(sidx.at[w], sidx_v)
        pltpu.sync_copy(didx.at[w], didx_v)
        pltpu.sync_copy(zeros128, accf.at[pl.ds(s * _RPT, _RPT)])
        plsc.subcore_barrier()

        def chunk(j, carry):
            cph = pltpu.async_copy(hgat.at[sidx_v.at[j]], hbuf, semh)
            _idx_build(sidx_v, didx_v, sidxb, didxb, j)
            cps = [pltpu.async_copy(asrcf.at[sidxb.at[h]], sabuf.at[h], sema)
                   for h in range(_HEADS)]
            cpd = [pltpu.async_copy(adstf.at[didxb.at[h]], dabuf.at[h], sema)
                   for h in range(_HEADS)]
            cph.wait()
            for cp in cps:
                cp.wait()
            for cp in cpd:
                cp.wait()

            def grp(t, cc):
                evs = []
                for h in range(_HEADS):
                    a = sabuf[h, pl.ds(t * 16, 16)] + dabuf[h, pl.ds(t * 16, 16)]
                    a = jnp.where(a > 0, a, a * 0.2)
                    evs.append(jnp.exp(a))
                for i in range(16):
                    e = t * 16 + i
                    for h in range(_HEADS):
                        sp = _splat(evs[h], i)
                        hbuf[e, pl.ds(16 * h, 16)] = (
                            hbuf[e, pl.ds(16 * h, 16)] * sp)
                return cc

            lax.fori_loop(0, _K // 16, grp, 0)
            pltpu.sync_copy(hbuf, accf.at[didx_v.at[j]], add=True)
            return carry

        lax.fori_loop(0, _C, chunk, 0)
        plsc.subcore_barrier()
        pltpu.sync_copy(accf.at[pl.ds(s * _RPT, _RPT)],
                        outf.at[c, pl.ds(s * _RPT, _RPT)])

    return k


def _make_gatz_kernel():
    out_t = [jax.ShapeDtypeStruct((2 * _ACC * _HEADS,), jnp.float32),
             jax.ShapeDtypeStruct((2 * _ACC,), jnp.float32)]

    @functools.partial(
        pl.kernel, mesh=_mesh, out_type=out_t,
        scratch_types=[
            pltpu.VMEM((_C, _K), jnp.int32),
            pltpu.VMEM((_C, _K), jnp.int32),
            pltpu.VMEM((_HEADS, _K), jnp.int32),
            pltpu.VMEM((_HEADS, _K), jnp.int32),
            pltpu.VMEM((_K,), jnp.int32),
            pltpu.VMEM((_K,), jnp.float32),
            pltpu.VMEM((_HEADS, _K), jnp.float32),
            pltpu.VMEM((_HEADS, _K), jnp.float32),
            pltpu.VMEM((_HEADS, _K), jnp.float32),
            pltpu.VMEM((_RPT * _HEADS,), jnp.float32),
            pltpu.VMEM_SHARED((_ACC * _HEADS,), jnp.float32),
            pltpu.VMEM_SHARED((_ACC,), jnp.float32),
            pltpu.SemaphoreType.DMA,
        ],
    )
    def k(asrcf, adstf, sidx, didx, zerosz, oute, outd,
          sidx_v, didx_v, sidxb, didxb, didxp, onesb, sabuf, dabuf, evbuf,
          zbuf, acce, accd, sema):
        c = lax.axis_index("c")
        s = lax.axis_index("s")
        w = c * 16 + s
        zr = _RPT * _HEADS
        pltpu.sync_copy(sidx.at[w], sidx_v)
        pltpu.sync_copy(didx.at[w], didx_v)
        pltpu.sync_copy(zerosz, zbuf)
        pltpu.sync_copy(zbuf, acce.at[pl.ds(s * zr, zr)])
        pltpu.sync_copy(zbuf.at[pl.ds(0, _RPT)], accd.at[pl.ds(s * _RPT, _RPT)])
        for t in range(_K // 16):
            onesb[pl.ds(t * 16, 16)] = jnp.ones((16,), jnp.float32)
        plsc.subcore_barrier()

        def chunk(j, carry):
            def tgrp(t, cc):
                sv = sidx_v[j, pl.ds(t * 16, 16)] * _HEADS
                dvp = didx_v[j, pl.ds(t * 16, 16)]
                didxp[pl.ds(t * 16, 16)] = dvp
                dv = dvp * _HEADS
                for h in range(_HEADS):
                    sidxb[h, pl.ds(t * 16, 16)] = sv + h
                    didxb[h, pl.ds(t * 16, 16)] = dv + h
                return cc

            lax.fori_loop(0, _K // 16, tgrp, 0)
            cps = [pltpu.async_copy(asrcf.at[sidxb.at[h]], sabuf.at[h], sema)
                   for h in range(_HEADS)]
            cpd = [pltpu.async_copy(adstf.at[didxb.at[h]], dabuf.at[h], sema)
                   for h in range(_HEADS)]
            pltpu.sync_copy(onesb, accd.at[didxp], add=True)
            for cp in cps:
                cp.wait()
            for cp in cpd:
                cp.wait()

            def grp(t, cc):
                for h in range(_HEADS):
                    a = sabuf[h, pl.ds(t * 16, 16)] + dabuf[h, pl.ds(t * 16, 16)]
                    a = jnp.where(a > 0, a, a * 0.2)
                    evbuf[h, pl.ds(t * 16, 16)] = jnp.exp(a)
                return cc

            lax.fori_loop(0, _K // 16, grp, 0)
            for h in range(_HEADS):
                pltpu.sync_copy(evbuf.at[h], acce.at[didxb.at[h]], add=True)
            return carry

        lax.fori_loop(0, _C, chunk, 0)
        plsc.subcore_barrier()
        pltpu.sync_copy(acce.at[pl.ds(s * zr, zr)], zbuf)
        pltpu.sync_copy(zbuf, oute.at[pl.ds(c * (_ACC * _HEADS) + s * zr, zr)])
        pltpu.sync_copy(accd.at[pl.ds(s * _RPT, _RPT)], zbuf.at[pl.ds(0, _RPT)])
        pltpu.sync_copy(zbuf.at[pl.ds(0, _RPT)],
                        outd.at[pl.ds(c * _ACC + s * _RPT, _RPT)])

    return k


_seg128_kernel = _make_seg_kernel(128)
_gatf_kernel = _make_gatf_kernel()
_gatz_kernel = _make_gatz_kernel()


def _bn(x, g, b, eps=1e-5):
    mu = jnp.mean(x, axis=0)
    var = jnp.var(x, axis=0)
    return g * (x - mu) / jnp.sqrt(var + eps) + b


def _leaky(a):
    return jnp.where(a > 0, a, 0.2 * a)


def _pad_rows(a):
    return jnp.concatenate(
        [a, jnp.zeros((_NROW - _N, a.shape[1]), a.dtype)], axis=0)


def _gat_dense(x, Wg, a_s, a_d):
    h = x @ Wg
    hr = h.reshape(_N, _HEADS, -1)
    asrc = jnp.sum(hr * a_s, axis=-1)
    adst = jnp.sum(hr * a_d, axis=-1)
    return h, asrc, adst


def _gat_combine(num, z, h, asrc, adst, bg):
    eself = jnp.exp(_leaky(asrc + adst))
    ch = h.shape[1] // _HEADS
    numr = num.reshape(_N, _HEADS, ch) + eself[:, :, None] * h.reshape(_N, _HEADS, ch)
    zr = z + eself
    out = numr / (zr[:, :, None] + 1e-16)
    return out.reshape(_N, -1) + bg


def _run_gat(x, Wg, a_s, a_d, bg, sidx, didx, zeros128, zerosz):
    h, asrc, adst = _gat_dense(x, Wg, a_s, a_d)
    hgat = _pad_rows(h)
    asrcf = _pad_rows(asrc).reshape(-1)
    adstf = _pad_rows(adst).reshape(-1)
    pf = _gatf_kernel(hgat, asrcf, adstf, sidx, didx, zeros128)
    pe, pdeg = _gatz_kernel(asrcf, adstf, sidx, didx, zerosz)
    num = pf[0, :_N] + pf[1, :_N]
    pe = pe.reshape(2, _ACC, _HEADS)
    z = pe[0, :_N] + pe[1, :_N]
    pdeg = pdeg.reshape(2, _ACC)
    indeg = pdeg[0, :_N] + pdeg[1, :_N]
    return _gat_combine(num, z, h, asrc, adst, bg), indeg


def _run_seg(table, sidx, didx, zeros):
    width = table.shape[1]
    if width < 128:
        table = jnp.concatenate(
            [table, jnp.zeros((_N, 128 - width), jnp.float32)], axis=1)
    parts = _seg128_kernel(_pad_rows(table), sidx, didx, zeros)
    return parts[0, :_N, :width] + parts[1, :_N, :width]


def _head_body(x_ref, batch_ref, *refs):
    (Wp, bp, Wa, ba, Wl1, bl1, gl1, bel1, Wl2, bl2, gl2, bel2,
     Wl3, bl3, Wl4, bl4, out_ref, xmax_ref) = refs
    x = x_ref[...]
    n = x.shape[0]
    b = batch_ref[...]
    gid = lax.broadcasted_iota(jnp.int32, (n, _NG), 1)
    oh = (b == gid).astype(jnp.float32)
    xsum = lax.dot_general(oh, x, (((0,), (0,)), ((), ())),
                           preferred_element_type=jnp.float32)
    cnt = jnp.sum(oh, axis=0)
    xmean = xsum / jnp.maximum(cnt, 1.0)[:, None]

    def gmax(g, carry):
        m = jnp.where(b == g, x, -jnp.inf)
        mg = jnp.max(m, axis=0)
        xmax_ref[pl.ds(g, 1), :] = mg[None, :]
        return carry

    lax.fori_loop(0, _NG, gmax, 0)
    xmax = xmax_ref[...]
    xmax = jnp.where(xmax > -1e30, xmax, 0.0)

    xp = jnp.concatenate([xmean, xmax, xsum], axis=1)
    xp = jax.nn.relu(xp @ Wp[...] + bp[...])
    att = jax.nn.sigmoid(xp @ Wa[...] + ba[...])
    xp = xp * att
    h = jax.nn.relu(xp @ Wl1[...] + bl1[...])
    h = _bn(h, gl1[...], bel1[...])
    h = jax.nn.relu(h @ Wl2[...] + bl2[...])
    h = _bn(h, gl2[...], bel2[...])
    h = jax.nn.relu(h @ Wl3[...] + bl3[...])
    h = h @ Wl4[...] + bl4[...]
    out_ref[...] = jax.nn.log_softmax(h, axis=1)


def _head(x4, batch, p):
    n = x4.shape[0]
    args = (x4, batch.astype(jnp.int32).reshape(n, 1),
            p['Wp'], p['bp'].reshape(1, -1), p['Wa'], p['ba'].reshape(1, -1),
            p['Wl1'], p['bl1'].reshape(1, -1), p['gl1'].reshape(1, -1),
            p['bel1'].reshape(1, -1),
            p['Wl2'], p['bl2'].reshape(1, -1), p['gl2'].reshape(1, -1),
            p['bel2'].reshape(1, -1),
            p['Wl3'], p['bl3'].reshape(1, -1), p['Wl4'], p['bl4'].reshape(1, -1))
    return pl.pallas_call(
        _head_body,
        out_shape=jax.ShapeDtypeStruct((_NG, 6), jnp.float32),
        scratch_shapes=[pltpu.VMEM((_NG, x4.shape[1]), jnp.float32)],
    )(*args)


def kernel(x, edge_index, batch, params):
    p = params

    pad = jnp.full((_EPAD - _E,), _N, jnp.int32)
    sidx = jnp.concatenate([edge_index[0].astype(jnp.int32), pad]).reshape(_NW, _C, _K)
    didx = jnp.concatenate([edge_index[1].astype(jnp.int32), pad]).reshape(_NW, _C, _K)
    zeros128 = jnp.zeros((_RPT, 128), jnp.float32)
    zerosz = jnp.zeros((_RPT * _HEADS,), jnp.float32)

    x0 = x @ p['W_in'] + p['b_in']
    x0 = jax.nn.relu(_bn(x0, p['g_in'], p['be_in']))

    xg1, indeg = _run_gat(x0, p['Wg1'], p['as1'], p['ad1'], p['bg1'],
                          sidx, didx, zeros128, zerosz)
    dinv = (indeg + 1.0) ** -0.5
    xg1 = jax.nn.elu(_bn(xg1, p['g1'], p['be1']))
    hc1 = dinv[:, None] * (x0 @ p['Wc1'])
    S1 = _run_seg(hc1, sidx, didx, zeros128)
    xc1 = dinv[:, None] * (S1 + hc1) + p['bc1']
    xc1 = jax.nn.elu(_bn(xc1, p['gc1'], p['bec1']))
    x1 = xg1 + xc1 + x0

    xg2, _unused = _run_gat(x1, p['Wg2'], p['as2'], p['ad2'], p['bg2'],
                            sidx, didx, zeros128, zerosz)
    xg2 = jax.nn.elu(_bn(xg2, p['g2'], p['be2']))
    hc2 = dinv[:, None] * (x1 @ p['Wc2'])
    S2 = _run_seg(hc2, sidx, didx, zeros128)
    xc2 = dinv[:, None] * (S2 + hc2) + p['bc2']
    xc2 = jax.nn.elu(_bn(xc2, p['gc2'], p['bec2']))
    x2 = xg2 + xc2 + x1

    agg = _run_seg(x2, sidx, didx, zeros128)
    x3 = agg @ p['Wr3'] + p['br3'] + x2 @ p['Wroot3']
    x3 = jax.nn.elu(_bn(x3, p['g3'], p['be3']))
    residual = x1 @ p['Wres'] + p['bres']

    hc4 = dinv[:, None] * (x3 @ p['W4'])
    S4 = _run_seg(hc4, sidx, didx, zeros128)
    x4 = dinv[:, None] * (S4 + hc4) + p['b4']
    x4 = jax.nn.elu(_bn(x4, p['g4'], p['be4']))
    x4 = x4 + residual

    return _head(x4, batch, p)
